# baseline (speedup 1.0000x reference)
import jax
import jax.numpy as jnp
from jax.experimental import pallas as pl

_SEARCH = 7
_PAD = (_SEARCH - 1) // 2
_KNN = 9
_NE = 2
_SD = _SEARCH * _SEARCH


def _make_body(R, H, W, stem, n_chan):
    def body(xp_ref, w_ref, out_ref):
        r = pl.program_id(1)
        row0 = r * R
        chans = [xp_ref[0, c, pl.ds(row0, R + 2 * _PAD), :] for c in range(n_chan)]

        def center(a):
            return a[_PAD:_PAD + R, _PAD:_PAD + W]

        offs = [(di, dj) for di in range(_SEARCH) for dj in range(_SEARCH)]
        fur_sl = [chans[0][di:di + R, dj:dj + W] for di, dj in offs]
        fup_sl = [[chans[2 + c][di:di + R, dj:dj + W] for di, dj in offs]
                  for c in range(3)]

        slots = []
        for e in range(_NE):
            npc = [center(chans[2 + 3 * e + c]) for c in range(3)]
            sk, sv = [], []
            for s in range(_SD):
                d0 = fup_sl[0][s] - npc[0]
                d1 = fup_sl[1][s] - npc[1]
                d2 = fup_sl[2][s] - npc[2]
                yk = d0 * d0 + d1 * d1 + d2 * d2
                yv = fur_sl[s]
                for j in range(len(sk)):
                    cmp = yk < sk[j]
                    nk = jnp.where(cmp, yk, sk[j])
                    nv = jnp.where(cmp, yv, sv[j])
                    yk = jnp.where(cmp, sk[j], yk)
                    yv = jnp.where(cmp, sv[j], yv)
                    sk[j] = nk
                    sv[j] = nv
                if len(sk) < _KNN:
                    sk.append(yk)
                    sv.append(yv)
            slots.extend(sv)
            slots.append(center(chans[e]))

        u = jnp.stack(slots, axis=0).reshape(_KNN * _NE + _NE, R * W)
        o = jax.lax.dot_general(
            w_ref[...], u, (((1,), (0,)), ((), ())),
            preferred_element_type=jnp.float32)
        o = o.reshape(stem, R, W)
        out_ref[0] = jnp.where(o >= 0, o, 0.01 * o)

    return body


def kernel(x, range_weight):
    B, C, H, W = x.shape
    stem = range_weight.shape[1]
    k_total = range_weight.shape[2]
    R = 8
    xp = jnp.pad(x, ((0, 0), (0, 0), (_PAD, _PAD), (_PAD, _PAD)))
    body = _make_body(R, H, W, stem, C)
    out = pl.pallas_call(
        body,
        grid=(B, H // R),
        in_specs=[
            pl.BlockSpec((1, C, H + 2 * _PAD, W + 2 * _PAD),
                         lambda b, r: (b, 0, 0, 0)),
            pl.BlockSpec((stem, k_total), lambda b, r: (0, 0)),
        ],
        out_specs=pl.BlockSpec((1, stem, R, W), lambda b, r: (b, 0, r, 0)),
        out_shape=jax.ShapeDtypeStruct((B, stem, H, W), jnp.float32),
    )(xp, range_weight[0])
    return out


# speedup vs baseline: 3.3551x; 3.3551x over previous
import jax
import jax.numpy as jnp
from jax.experimental import pallas as pl
from jax.experimental.pallas import tpu as pltpu

_SEARCH = 7
_PAD = (_SEARCH - 1) // 2
_KNN = 9
_NE = 2
_SD = _SEARCH * _SEARCH


def _tree_sum(xs):
    xs = list(xs)
    while len(xs) > 1:
        nxt = [xs[i] + xs[i + 1] for i in range(0, len(xs) - 1, 2)]
        if len(xs) % 2:
            nxt.append(xs[-1])
        xs = nxt
    return xs[0]


def _make_body(R, H, W, stem, n_chan):
    RW = R + 2 * _PAD
    offs = [(di, dj) for di in range(_SEARCH) for dj in range(_SEARCH)]

    def body(xp_ref, w_ref, out_ref, fur_ref, fp0_ref, fp1_ref, fp2_ref,
             key_ref, cen_ref):
        r = pl.program_id(1)
        row0 = r * R

        chans = {c: xp_ref[0, c, pl.ds(row0, RW), :] for c in range(n_chan)}

        for s, (di, dj) in enumerate(offs):
            fur_ref[s] = chans[0][di:di + R, dj:dj + W]
            fp0_ref[s] = chans[2][di:di + R, dj:dj + W]
            fp1_ref[s] = chans[3][di:di + R, dj:dj + W]
            fp2_ref[s] = chans[4][di:di + R, dj:dj + W]
        for c in range(n_chan):
            cen_ref[c] = chans[c][_PAD:_PAD + R, _PAD:_PAD + W]

        slots = []
        for e in range(_NE):
            npc = [cen_ref[2 + 3 * e + c] for c in range(3)]
            sk = []
            for s in range(_SD):
                d0 = fp0_ref[s] - npc[0]
                d1 = fp1_ref[s] - npc[1]
                d2 = fp2_ref[s] - npc[2]
                yk = d0 * d0 + d1 * d1 + d2 * d2
                key_ref[s] = yk
                for j in range(len(sk)):
                    lo = jnp.minimum(yk, sk[j])
                    yk = jnp.maximum(yk, sk[j])
                    sk[j] = lo
                if len(sk) < _KNN:
                    sk.append(yk)
            for k in range(_KNN):
                terms = []
                for s in range(_SD):
                    m = key_ref[s] == sk[k]
                    terms.append(jnp.where(m, fur_ref[s], 0.0))
                slots.append(_tree_sum(terms))
            slots.append(cen_ref[e])

        u = jnp.stack(slots, axis=0).reshape(_KNN * _NE + _NE, R * W)
        o = jax.lax.dot_general(
            w_ref[...], u, (((1,), (0,)), ((), ())),
            preferred_element_type=jnp.float32)
        o = o.reshape(stem, R, W)
        out_ref[0] = jnp.where(o >= 0, o, 0.01 * o)

    return body


def kernel(x, range_weight):
    B, C, H, W = x.shape
    stem = range_weight.shape[1]
    k_total = range_weight.shape[2]
    R = 16
    xp = jnp.pad(x, ((0, 0), (0, 0), (_PAD, _PAD), (_PAD, _PAD)))
    body = _make_body(R, H, W, stem, C)
    win_scr = pltpu.VMEM((_SD, R, W), jnp.float32)
    out = pl.pallas_call(
        body,
        grid=(B, H // R),
        in_specs=[
            pl.BlockSpec((1, C, H + 2 * _PAD, W + 2 * _PAD),
                         lambda b, r: (b, 0, 0, 0)),
            pl.BlockSpec((stem, k_total), lambda b, r: (0, 0)),
        ],
        out_specs=pl.BlockSpec((1, stem, R, W), lambda b, r: (b, 0, r, 0)),
        out_shape=jax.ShapeDtypeStruct((B, stem, H, W), jnp.float32),
        scratch_shapes=[
            win_scr, win_scr, win_scr, win_scr, win_scr,
            pltpu.VMEM((8, R, W), jnp.float32),
        ],
    )(xp, range_weight[0])
    return out


# speedup vs baseline: 3.3839x; 1.0086x over previous
import jax
import jax.numpy as jnp
from jax.experimental import pallas as pl
from jax.experimental.pallas import tpu as pltpu

_SEARCH = 7
_PAD = (_SEARCH - 1) // 2
_KNN = 9
_NE = 2
_SD = _SEARCH * _SEARCH


def _tree_sum(xs):
    xs = list(xs)
    while len(xs) > 1:
        nxt = [xs[i] + xs[i + 1] for i in range(0, len(xs) - 1, 2)]
        if len(xs) % 2:
            nxt.append(xs[-1])
        xs = nxt
    return xs[0]


def _make_body(R, H, W, stem, n_chan):
    RW = R + 2 * _PAD
    offs = [(di, dj) for di in range(_SEARCH) for dj in range(_SEARCH)]

    def body(xp_ref, w_ref, out_ref, fur_ref, fp0_ref, fp1_ref, fp2_ref,
             key_ref, cen_ref):
        r = pl.program_id(1)
        row0 = r * R

        chans = {c: xp_ref[0, c, pl.ds(row0, RW), :] for c in range(n_chan)}

        for s, (di, dj) in enumerate(offs):
            fur_ref[s] = chans[0][di:di + R, dj:dj + W]
            fp0_ref[s] = chans[2][di:di + R, dj:dj + W]
            fp1_ref[s] = chans[3][di:di + R, dj:dj + W]
            fp2_ref[s] = chans[4][di:di + R, dj:dj + W]
        for c in range(n_chan):
            cen_ref[c] = chans[c][_PAD:_PAD + R, _PAD:_PAD + W]

        slots = []
        for e in range(_NE):
            npc = [cen_ref[2 + 3 * e + c] for c in range(3)]
            sk = []
            for s in range(_SD):
                d0 = fp0_ref[s] - npc[0]
                d1 = fp1_ref[s] - npc[1]
                d2 = fp2_ref[s] - npc[2]
                yk = d0 * d0 + d1 * d1 + d2 * d2
                key_ref[s] = yk
                for j in range(len(sk)):
                    lo = jnp.minimum(yk, sk[j])
                    yk = jnp.maximum(yk, sk[j])
                    sk[j] = lo
                if len(sk) < _KNN:
                    sk.append(yk)
            for k in range(_KNN):
                terms = []
                for s in range(_SD):
                    m = key_ref[s] == sk[k]
                    terms.append(jnp.where(m, fur_ref[s], 0.0))
                slots.append(_tree_sum(terms))
            slots.append(cen_ref[e])

        u = jnp.stack(slots, axis=0).reshape(_KNN * _NE + _NE, R * W)
        o = jax.lax.dot_general(
            w_ref[...], u, (((1,), (0,)), ((), ())),
            preferred_element_type=jnp.float32)
        o = o.reshape(stem, R, W)
        out_ref[0] = jnp.where(o >= 0, o, 0.01 * o)

    return body


def kernel(x, range_weight):
    B, C, H, W = x.shape
    stem = range_weight.shape[1]
    k_total = range_weight.shape[2]
    R = 32
    xp = jnp.pad(x, ((0, 0), (0, 0), (_PAD, _PAD), (_PAD, _PAD)))
    body = _make_body(R, H, W, stem, C)
    win_scr = pltpu.VMEM((_SD, R, W), jnp.float32)
    out = pl.pallas_call(
        body,
        grid=(B, H // R),
        in_specs=[
            pl.BlockSpec((1, C, H + 2 * _PAD, W + 2 * _PAD),
                         lambda b, r: (b, 0, 0, 0)),
            pl.BlockSpec((stem, k_total), lambda b, r: (0, 0)),
        ],
        out_specs=pl.BlockSpec((1, stem, R, W), lambda b, r: (b, 0, r, 0)),
        out_shape=jax.ShapeDtypeStruct((B, stem, H, W), jnp.float32),
        scratch_shapes=[
            win_scr, win_scr, win_scr, win_scr, win_scr,
            pltpu.VMEM((8, R, W), jnp.float32),
        ],
    )(xp, range_weight[0])
    return out


# speedup vs baseline: 3.5858x; 1.0597x over previous
import jax
import jax.numpy as jnp
from jax.experimental import pallas as pl
from jax.experimental.pallas import tpu as pltpu

_SEARCH = 7
_PAD = (_SEARCH - 1) // 2
_KNN = 9
_NE = 2
_SD = _SEARCH * _SEARCH
TR = 8


def _make_body(R, H, W, stem, n_chan):
    RW = R + 2 * _PAD
    offs = [(di, dj) for di in range(_SEARCH) for dj in range(_SEARCH)]

    def body(xp_ref, w_ref, out_ref, fur_ref, fp0_ref, fp1_ref, fp2_ref,
             q_ref, cen_ref):
        r = pl.program_id(1)
        row0 = r * R

        chans = {c: xp_ref[0, c, pl.ds(row0, RW), :] for c in range(n_chan)}
        qmap = (chans[2] * chans[2] + chans[3] * chans[3]
                + chans[4] * chans[4])

        for s, (di, dj) in enumerate(offs):
            fur_ref[s] = chans[0][di:di + R, dj:dj + W]
            fp0_ref[s] = chans[2][di:di + R, dj:dj + W]
            fp1_ref[s] = chans[3][di:di + R, dj:dj + W]
            fp2_ref[s] = chans[4][di:di + R, dj:dj + W]
            q_ref[s] = qmap[di:di + R, dj:dj + W]
        for c in range(n_chan):
            cen_ref[c] = chans[c][_PAD:_PAD + R, _PAD:_PAD + W]

        for t in range(R // TR):
            tr0 = t * TR
            slots = []
            for e in range(_NE):
                t0 = 2.0 * cen_ref[2 + 3 * e, pl.ds(tr0, TR), :]
                t1 = 2.0 * cen_ref[3 + 3 * e, pl.ds(tr0, TR), :]
                t2 = 2.0 * cen_ref[4 + 3 * e, pl.ds(tr0, TR), :]
                sk = []
                sv = []
                for s in range(_SD):
                    yk = q_ref[s, pl.ds(tr0, TR), :] - (
                        fp0_ref[s, pl.ds(tr0, TR), :] * t0
                        + fp1_ref[s, pl.ds(tr0, TR), :] * t1
                        + fp2_ref[s, pl.ds(tr0, TR), :] * t2)
                    yv = fur_ref[s, pl.ds(tr0, TR), :]
                    for j in range(len(sk)):
                        m = yk < sk[j]
                        klo = jnp.minimum(yk, sk[j])
                        yk = jnp.maximum(yk, sk[j])
                        sk[j] = klo
                        vlo = jnp.where(m, yv, sv[j])
                        yv = jnp.where(m, sv[j], yv)
                        sv[j] = vlo
                    if len(sk) < _KNN:
                        sk.append(yk)
                        sv.append(yv)
                slots.extend(sv)
                slots.append(cen_ref[e, pl.ds(tr0, TR), :])

            u = jnp.stack(slots, axis=0).reshape(_KNN * _NE + _NE, TR * W)
            o = jax.lax.dot_general(
                w_ref[...], u, (((1,), (0,)), ((), ())),
                preferred_element_type=jnp.float32)
            o = o.reshape(stem, TR, W)
            out_ref[0, :, pl.ds(tr0, TR), :] = jnp.where(o >= 0, o, 0.01 * o)

    return body


def kernel(x, range_weight):
    B, C, H, W = x.shape
    stem = range_weight.shape[1]
    k_total = range_weight.shape[2]
    R = 32
    xp = jnp.pad(x, ((0, 0), (0, 0), (_PAD, _PAD), (_PAD, _PAD)))
    body = _make_body(R, H, W, stem, C)
    win_scr = pltpu.VMEM((_SD, R, W), jnp.float32)
    out = pl.pallas_call(
        body,
        grid=(B, H // R),
        in_specs=[
            pl.BlockSpec((1, C, H + 2 * _PAD, W + 2 * _PAD),
                         lambda b, r: (b, 0, 0, 0)),
            pl.BlockSpec((stem, k_total), lambda b, r: (0, 0)),
        ],
        out_specs=pl.BlockSpec((1, stem, R, W), lambda b, r: (b, 0, r, 0)),
        out_shape=jax.ShapeDtypeStruct((B, stem, H, W), jnp.float32),
        scratch_shapes=[
            win_scr, win_scr, win_scr, win_scr, win_scr,
            pltpu.VMEM((8, R, W), jnp.float32),
        ],
    )(xp, range_weight[0])
    return out


# speedup vs baseline: 3.6200x; 1.0096x over previous
import jax
import jax.numpy as jnp
from jax.experimental import pallas as pl
from jax.experimental.pallas import tpu as pltpu

_SEARCH = 7
_PAD = (_SEARCH - 1) // 2
_KNN = 9
_NE = 2
_SD = _SEARCH * _SEARCH
_TR = 8
_TL = 128


def _make_body(R, W, WP, stem, n_chan):
    RW = R + 2 * _PAD
    offs = [(di, dj) for di in range(_SEARCH) for dj in range(_SEARCH)]

    def body(xp_ref, w_ref, out_ref, fur_ref, fp0_ref, fp1_ref, fp2_ref,
             q_ref, cen_ref):
        r = pl.program_id(1)
        row0 = r * R

        chans = {c: xp_ref[0, c, pl.ds(row0, RW), :] for c in range(n_chan)}
        qmap = (chans[2] * chans[2] + chans[3] * chans[3]
                + chans[4] * chans[4])

        for s, (di, dj) in enumerate(offs):
            fur_ref[s] = chans[0][di:di + R, dj:dj + WP]
            fp0_ref[s] = chans[2][di:di + R, dj:dj + WP]
            fp1_ref[s] = chans[3][di:di + R, dj:dj + WP]
            fp2_ref[s] = chans[4][di:di + R, dj:dj + WP]
            q_ref[s] = qmap[di:di + R, dj:dj + WP]
        for c in range(n_chan):
            cen_ref[c] = chans[c][_PAD:_PAD + R, _PAD:_PAD + WP]

        for t in range(R // _TR):
            tr0 = t * _TR
            for lt in range(WP // _TL):
                lc0 = lt * _TL
                if lc0 >= W:
                    continue
                slots = []
                for e in range(_NE):
                    t0 = 2.0 * cen_ref[2 + 3 * e, pl.ds(tr0, _TR),
                                       pl.ds(lc0, _TL)]
                    t1 = 2.0 * cen_ref[3 + 3 * e, pl.ds(tr0, _TR),
                                       pl.ds(lc0, _TL)]
                    t2 = 2.0 * cen_ref[4 + 3 * e, pl.ds(tr0, _TR),
                                       pl.ds(lc0, _TL)]
                    sk = []
                    sv = []
                    for s in range(_SD):
                        yk = q_ref[s, pl.ds(tr0, _TR), pl.ds(lc0, _TL)] - (
                            fp0_ref[s, pl.ds(tr0, _TR), pl.ds(lc0, _TL)] * t0
                            + fp1_ref[s, pl.ds(tr0, _TR), pl.ds(lc0, _TL)] * t1
                            + fp2_ref[s, pl.ds(tr0, _TR), pl.ds(lc0, _TL)] * t2)
                        yv = fur_ref[s, pl.ds(tr0, _TR), pl.ds(lc0, _TL)]
                        for j in range(len(sk)):
                            m = yk < sk[j]
                            klo = jnp.minimum(yk, sk[j])
                            yk = jnp.maximum(yk, sk[j])
                            sk[j] = klo
                            vlo = jnp.where(m, yv, sv[j])
                            yv = jnp.where(m, sv[j], yv)
                            sv[j] = vlo
                        if len(sk) < _KNN:
                            sk.append(yk)
                            sv.append(yv)
                    slots.extend(sv)
                    slots.append(cen_ref[e, pl.ds(tr0, _TR), pl.ds(lc0, _TL)])

                u = jnp.stack(slots, axis=0).reshape(_KNN * _NE + _NE,
                                                     _TR * _TL)
                o = jax.lax.dot_general(
                    w_ref[...], u, (((1,), (0,)), ((), ())),
                    preferred_element_type=jnp.float32)
                o = o.reshape(stem, _TR, _TL)
                ww = min(_TL, W - lc0)
                out_ref[0, :, pl.ds(tr0, _TR), pl.ds(lc0, ww)] = (
                    jnp.where(o >= 0, o, 0.01 * o)[:, :, :ww])

    return body


def kernel(x, range_weight):
    B, C, H, W = x.shape
    stem = range_weight.shape[1]
    k_total = range_weight.shape[2]
    R = 32
    WP = ((W + _TL - 1) // _TL) * _TL
    xp = jnp.pad(x, ((0, 0), (0, 0), (_PAD, _PAD),
                     (_PAD, WP + _PAD - W)))
    body = _make_body(R, W, WP, stem, C)
    win_scr = pltpu.VMEM((_SD, R, WP), jnp.float32)
    out = pl.pallas_call(
        body,
        grid=(B, H // R),
        in_specs=[
            pl.BlockSpec((1, C, H + 2 * _PAD, WP + 2 * _PAD),
                         lambda b, r: (b, 0, 0, 0)),
            pl.BlockSpec((stem, k_total), lambda b, r: (0, 0)),
        ],
        out_specs=pl.BlockSpec((1, stem, R, W), lambda b, r: (b, 0, r, 0)),
        out_shape=jax.ShapeDtypeStruct((B, stem, H, W), jnp.float32),
        scratch_shapes=[
            win_scr, win_scr, win_scr, win_scr, win_scr,
            pltpu.VMEM((8, R, WP), jnp.float32),
        ],
    )(xp, range_weight[0])
    return out


# speedup vs baseline: 4.2796x; 1.1822x over previous
import jax
import jax.numpy as jnp
from jax.experimental import pallas as pl
from jax.experimental.pallas import tpu as pltpu

_SEARCH = 7
_PAD = (_SEARCH - 1) // 2
_KNN = 9
_NE = 2
_SD = _SEARCH * _SEARCH
_TR = 8
_TL = 128

_SORT7 = [(1, 2), (3, 4), (5, 6),
          (0, 2), (3, 5), (4, 6),
          (0, 1), (4, 5), (2, 6),
          (0, 4), (1, 5),
          (0, 3), (2, 5),
          (1, 3), (2, 4),
          (2, 3)]

_MERGE16_9 = (
    [(i, i + 8) for i in range(8)]
    + [(i, i + 4) for i in (0, 1, 2, 3, 8, 9, 10, 11)]
    + [(i, i + 2) for i in (0, 1, 4, 5, 8, 9)]
    + [(i, i + 1) for i in (0, 2, 4, 6, 8)])


def _ce(k, v, i, j):
    ki, kj = k[i], k[j]
    m = kj < ki
    k[i] = jnp.minimum(ki, kj)
    k[j] = jnp.maximum(ki, kj)
    vi, vj = v[i], v[j]
    v[i] = jnp.where(m, vj, vi)
    v[j] = jnp.where(m, vi, vj)


def _make_body(R, W, WP, stem, n_chan):
    RW = R + 2 * _PAD
    offs = [(di, dj) for di in range(_SEARCH) for dj in range(_SEARCH)]

    def body(xp_ref, w_ref, out_ref, fur_ref, fp0_ref, fp1_ref, fp2_ref,
             q_ref, cen_ref):
        r = pl.program_id(1)
        row0 = r * R

        chans = {c: xp_ref[0, c, pl.ds(row0, RW), :] for c in range(n_chan)}
        qmap = (chans[2] * chans[2] + chans[3] * chans[3]
                + chans[4] * chans[4])

        for s, (di, dj) in enumerate(offs):
            fur_ref[s] = chans[0][di:di + R, dj:dj + WP]
            fp0_ref[s] = chans[2][di:di + R, dj:dj + WP]
            fp1_ref[s] = chans[3][di:di + R, dj:dj + WP]
            fp2_ref[s] = chans[4][di:di + R, dj:dj + WP]
            q_ref[s] = qmap[di:di + R, dj:dj + WP]
        for c in range(n_chan):
            cen_ref[c] = chans[c][_PAD:_PAD + R, _PAD:_PAD + WP]

        for t in range(R // _TR):
            tr0 = t * _TR
            for lt in range(WP // _TL):
                lc0 = lt * _TL
                if lc0 >= W:
                    continue
                slots = []
                for e in range(_NE):
                    t0 = 2.0 * cen_ref[2 + 3 * e, pl.ds(tr0, _TR),
                                       pl.ds(lc0, _TL)]
                    t1 = 2.0 * cen_ref[3 + 3 * e, pl.ds(tr0, _TR),
                                       pl.ds(lc0, _TL)]
                    t2 = 2.0 * cen_ref[4 + 3 * e, pl.ds(tr0, _TR),
                                       pl.ds(lc0, _TL)]
                    ak = None
                    av = None
                    for g in range(_SD // _SEARCH):
                        gk = []
                        gv = []
                        for w in range(_SEARCH):
                            s = g * _SEARCH + w
                            yk = q_ref[s, pl.ds(tr0, _TR),
                                       pl.ds(lc0, _TL)] - (
                                fp0_ref[s, pl.ds(tr0, _TR),
                                        pl.ds(lc0, _TL)] * t0
                                + fp1_ref[s, pl.ds(tr0, _TR),
                                          pl.ds(lc0, _TL)] * t1
                                + fp2_ref[s, pl.ds(tr0, _TR),
                                          pl.ds(lc0, _TL)] * t2)
                            gk.append(yk)
                            gv.append(fur_ref[s, pl.ds(tr0, _TR),
                                              pl.ds(lc0, _TL)])
                        for i, j in _SORT7:
                            _ce(gk, gv, i, j)
                        if ak is None:
                            inf = jnp.full((_TR, _TL), jnp.inf, jnp.float32)
                            ak = gk + [inf, inf]
                            av = gv + [gv[0], gv[0]]
                        else:
                            mk = ak + gk[::-1]
                            mv = av + gv[::-1]
                            for i, j in _MERGE16_9:
                                _ce(mk, mv, i, j)
                            ak = mk[:_KNN]
                            av = mv[:_KNN]
                    slots.extend(av)
                    slots.append(cen_ref[e, pl.ds(tr0, _TR), pl.ds(lc0, _TL)])

                u = jnp.stack(slots, axis=0).reshape(_KNN * _NE + _NE,
                                                     _TR * _TL)
                o = jax.lax.dot_general(
                    w_ref[...], u, (((1,), (0,)), ((), ())),
                    preferred_element_type=jnp.float32)
                o = o.reshape(stem, _TR, _TL)
                ww = min(_TL, W - lc0)
                out_ref[0, :, pl.ds(tr0, _TR), pl.ds(lc0, ww)] = (
                    jnp.where(o >= 0, o, 0.01 * o)[:, :, :ww])

    return body


def kernel(x, range_weight):
    B, C, H, W = x.shape
    stem = range_weight.shape[1]
    k_total = range_weight.shape[2]
    R = 32
    WP = ((W + _TL - 1) // _TL) * _TL
    xp = jnp.pad(x, ((0, 0), (0, 0), (_PAD, _PAD),
                     (_PAD, WP + _PAD - W)))
    body = _make_body(R, W, WP, stem, C)
    win_scr = pltpu.VMEM((_SD, R, WP), jnp.float32)
    out = pl.pallas_call(
        body,
        grid=(B, H // R),
        in_specs=[
            pl.BlockSpec((1, C, H + 2 * _PAD, WP + 2 * _PAD),
                         lambda b, r: (b, 0, 0, 0)),
            pl.BlockSpec((stem, k_total), lambda b, r: (0, 0)),
        ],
        out_specs=pl.BlockSpec((1, stem, R, W), lambda b, r: (b, 0, r, 0)),
        out_shape=jax.ShapeDtypeStruct((B, stem, H, W), jnp.float32),
        scratch_shapes=[
            win_scr, win_scr, win_scr, win_scr, win_scr,
            pltpu.VMEM((8, R, WP), jnp.float32),
        ],
    )(xp, range_weight[0])
    return out
